# Initial kernel scaffold; baseline (speedup 1.0000x reference)
import jax
import jax.numpy as jnp
from jax.experimental import pallas as pl


def kernel(x, edge_index, batch_vector, W0, as0, ad0, b0, W1, as1, ad1, b1, W2, as2, ad2, b2, lin_W, lin_b):
    raise NotImplementedError("write your pallas kernel here")


# speedup vs baseline: 11.4201x; 11.4201x over previous
import functools

import jax
import jax.numpy as jnp
from jax.experimental import pallas as pl
from jax.experimental.pallas import tpu as pltpu

N = 10000
DF = 128
H = 8
HID = 16
G = 512
NB = 200
NBLK = N // NB
CE = 1024
NT = 2000


def _transform_kernel(h_ref, w_ref, as_ref, ad_ref, hnew_ref, asrc_ref, adst_ref):
    h = h_ref[...]
    hn = jnp.dot(h, w_ref[...], preferred_element_type=jnp.float32)
    hnew_ref[...] = hn
    asrc_ref[...] = jnp.dot(hn, as_ref[...], preferred_element_type=jnp.float32)
    adst_ref[...] = jnp.dot(hn, ad_ref[...], preferred_element_type=jnp.float32)


def _edge_kernel(off_ref, hsrc_hbm, asrce_hbm, dst_hbm, adst_ref, c_ref, b_ref,
                 r_ref, out_ref, h_scr, a_scr, d_scr, sem_h, sem_a, sem_d):
    n = pl.program_id(0)
    start = off_ref[n]
    end = off_ref[n + 1]
    nchunks = (end - start + CE - 1) // CE
    adst_blk = adst_ref[...]
    c = c_ref[...]
    r = r_ref[...]
    base = n * NB
    iota = jax.lax.broadcasted_iota(jnp.int32, (CE, NB), 1)

    def body(k, carry):
        s_acc, den_acc = carry
        s0 = start + k * CE
        cp_h = pltpu.make_async_copy(hsrc_hbm.at[pl.ds(s0, CE), :], h_scr, sem_h)
        cp_a = pltpu.make_async_copy(asrce_hbm.at[pl.ds(s0, CE), :], a_scr, sem_a)
        cp_d = pltpu.make_async_copy(dst_hbm.at[pl.ds(s0, CE), :], d_scr, sem_d)
        cp_h.start()
        cp_a.start()
        cp_d.start()
        cp_h.wait()
        cp_a.wait()
        cp_d.wait()
        local = d_scr[...] - base
        oh = (iota == local).astype(jnp.float32)
        adst_e = jnp.dot(oh, adst_blk, preferred_element_type=jnp.float32)
        in_blk = (local >= 0) & (local < NB)
        alpha = a_scr[...] + adst_e
        alpha = jnp.where(alpha >= 0, alpha, 0.2 * alpha)
        ea = jnp.where(in_blk, jnp.exp(alpha - c), 0.0)
        ea_rep = jnp.dot(ea, r, preferred_element_type=jnp.float32)
        msg = h_scr[...] * ea_rep
        s_acc = s_acc + jax.lax.dot_general(
            oh, msg, (((0,), (0,)), ((), ())), preferred_element_type=jnp.float32)
        den_acc = den_acc + jax.lax.dot_general(
            oh, ea, (((0,), (0,)), ((), ())), preferred_element_type=jnp.float32)
        return s_acc, den_acc

    s0 = jnp.zeros((NB, DF), jnp.float32)
    d0 = jnp.zeros((NB, H), jnp.float32)
    s_sum, den = jax.lax.fori_loop(0, nchunks, body, (s0, d0))
    den_rep = jnp.dot(jnp.maximum(den, 1e-16), r_ref[...],
                      preferred_element_type=jnp.float32)
    out = s_sum / den_rep + b_ref[...]
    out_ref[...] = jnp.where(out > 0, out, jnp.exp(jnp.minimum(out, 0.0)) - 1.0)


def _pool_kernel(h_ref, bv_ref, linw_ref, linb_ref, out_ref, s_scr, c_scr):
    i = pl.program_id(0)

    @pl.when(i == 0)
    def _():
        s_scr[...] = jnp.zeros_like(s_scr)
        c_scr[...] = jnp.zeros_like(c_scr)

    iota = jax.lax.broadcasted_iota(jnp.int32, (NT, G), 1)
    oh = (iota == bv_ref[...]).astype(jnp.float32)
    s_scr[...] += jax.lax.dot_general(
        oh, h_ref[...], (((0,), (0,)), ((), ())), preferred_element_type=jnp.float32)
    c_scr[...] += jax.lax.dot_general(
        oh, jnp.ones((NT, 1), jnp.float32), (((0,), (0,)), ((), ())),
        preferred_element_type=jnp.float32)

    @pl.when(i == pl.num_programs(0) - 1)
    def _():
        mean = s_scr[...] / jnp.maximum(c_scr[...], 1.0)
        out_ref[...] = jnp.dot(mean, linw_ref[...],
                               preferred_element_type=jnp.float32) + linb_ref[...]


def _block_diag_attn(a):
    rows = jnp.arange(H * HID)
    cols = rows // HID
    return jnp.zeros((H * HID, H), jnp.float32).at[rows, cols].set(a.reshape(-1))


_transform = pl.pallas_call(
    _transform_kernel,
    grid=(N // NT,),
    in_specs=[
        pl.BlockSpec((NT, DF), lambda i: (i, 0)),
        pl.BlockSpec((DF, DF), lambda i: (0, 0)),
        pl.BlockSpec((DF, H), lambda i: (0, 0)),
        pl.BlockSpec((DF, H), lambda i: (0, 0)),
    ],
    out_specs=[
        pl.BlockSpec((NT, DF), lambda i: (i, 0)),
        pl.BlockSpec((NT, H), lambda i: (i, 0)),
        pl.BlockSpec((NT, H), lambda i: (i, 0)),
    ],
    out_shape=[
        jax.ShapeDtypeStruct((N, DF), jnp.float32),
        jax.ShapeDtypeStruct((N, H), jnp.float32),
        jax.ShapeDtypeStruct((N, H), jnp.float32),
    ],
)

_edge_call = pl.pallas_call(
    _edge_kernel,
    grid_spec=pltpu.PrefetchScalarGridSpec(
        num_scalar_prefetch=1,
        grid=(NBLK,),
        in_specs=[
            pl.BlockSpec(memory_space=pltpu.MemorySpace.HBM),
            pl.BlockSpec(memory_space=pltpu.MemorySpace.HBM),
            pl.BlockSpec(memory_space=pltpu.MemorySpace.HBM),
            pl.BlockSpec((NB, H), lambda n, off: (n, 0)),
            pl.BlockSpec((1, H), lambda n, off: (0, 0)),
            pl.BlockSpec((1, DF), lambda n, off: (0, 0)),
            pl.BlockSpec((H, DF), lambda n, off: (0, 0)),
        ],
        out_specs=pl.BlockSpec((NB, DF), lambda n, off: (n, 0)),
        scratch_shapes=[
            pltpu.VMEM((CE, DF), jnp.float32),
            pltpu.VMEM((CE, H), jnp.float32),
            pltpu.VMEM((CE, 1), jnp.int32),
            pltpu.SemaphoreType.DMA,
            pltpu.SemaphoreType.DMA,
            pltpu.SemaphoreType.DMA,
        ],
    ),
    out_shape=jax.ShapeDtypeStruct((N, DF), jnp.float32),
)

_pool_call = pl.pallas_call(
    _pool_kernel,
    grid=(N // NT,),
    in_specs=[
        pl.BlockSpec((NT, DF), lambda i: (i, 0)),
        pl.BlockSpec((NT, 1), lambda i: (i, 0)),
        pl.BlockSpec((DF, 1), lambda i: (0, 0)),
        pl.BlockSpec((1, 1), lambda i: (0, 0)),
    ],
    out_specs=pl.BlockSpec((G, 1), lambda i: (0, 0)),
    out_shape=jax.ShapeDtypeStruct((G, 1), jnp.float32),
    scratch_shapes=[
        pltpu.VMEM((G, DF), jnp.float32),
        pltpu.VMEM((G, 1), jnp.float32),
    ],
)


@functools.partial(jax.jit)
def kernel(x, edge_index, batch_vector, W0, as0, ad0, b0, W1, as1, ad1, b1,
           W2, as2, ad2, b2, lin_W, lin_b):
    loops = jnp.arange(N, dtype=edge_index.dtype)
    src = jnp.concatenate([edge_index[0], loops])
    dst = jnp.concatenate([edge_index[1], loops])
    ee = src.shape[0]
    ep = ((ee + CE - 1) // CE + 1) * CE
    perm = jnp.argsort(dst)
    dst_s = dst[perm]
    src_s = src[perm]
    dst_p = jnp.concatenate([dst_s, jnp.full((ep - ee,), N, jnp.int32)])
    src_p = jnp.concatenate([src_s, jnp.zeros((ep - ee,), jnp.int32)])
    off = jnp.searchsorted(dst_p, jnp.arange(0, N + 1, NB)).astype(jnp.int32)
    dst2d = dst_p.reshape(ep, 1)

    rrows = jnp.arange(DF)
    rmat = jnp.zeros((H, DF), jnp.float32).at[rrows // HID, rrows].set(1.0)

    h = x
    for W, a_s, a_d, b in ((W0, as0, ad0, b0), (W1, as1, ad1, b1),
                           (W2, as2, ad2, b2)):
        hn, asrc, adst = _transform(h, W, _block_diag_attn(a_s),
                                    _block_diag_attn(a_d))
        cmax = jnp.max(asrc, axis=0) + jnp.max(adst, axis=0)
        cmax = jnp.where(cmax >= 0, cmax, 0.2 * cmax).reshape(1, H)
        asrc_e = jnp.take(asrc, src_p, axis=0)
        hsrc_e = jnp.take(hn, src_p, axis=0)
        h = _edge_call(off, hsrc_e, asrc_e, dst2d, adst, cmax,
                       b.reshape(1, DF), rmat)

    return _pool_call(h, batch_vector.reshape(N, 1), lin_W,
                      lin_b.reshape(1, 1))
